# baseline (speedup 1.0000x reference)
import functools

import jax
import jax.numpy as jnp
from jax.experimental import pallas as pl
from jax.experimental.pallas import tpu as pltpu

T = 128
D = 768
H = 1536
E = 64


def _moe_kernel(x_ref, gw_ref, gb_ref, w1_ref, b1_ref, w2_ref, b2_ref,
                out_ref, aux_ref, gatew_ref):
    e = pl.program_id(0)

    @pl.when(e == 0)
    def _():
        x = x_ref[...]
        logits = jax.lax.dot_general(
            x.astype(jnp.bfloat16), gw_ref[...].astype(jnp.bfloat16),
            (((1,), (0,)), ((), ())),
            preferred_element_type=jnp.float32,
        ) + gb_ref[...]
        m = jnp.max(logits, axis=1, keepdims=True)
        ex = jnp.exp(logits - m)
        probs = 0.99 * (ex / jnp.sum(ex, axis=1, keepdims=True)) + 0.01 / E

        iota = jax.lax.broadcasted_iota(jnp.int32, (T, E), 1)
        m1 = jnp.max(probs, axis=1, keepdims=True)
        i1 = jnp.min(jnp.where(probs == m1, iota, E), axis=1, keepdims=True)
        masked = jnp.where(iota == i1, -1.0, probs)
        m2 = jnp.max(masked, axis=1, keepdims=True)
        i2 = jnp.min(jnp.where(masked == m2, iota, E), axis=1, keepdims=True)
        s = m1 + m2
        gatew_ref[...] = jnp.where(iota == i1, m1 / s, 0.0) + jnp.where(
            iota == i2, m2 / s, 0.0)
        aux = jnp.sum(probs * probs) * (E / T)
        aux_ref[...] = jnp.full((8, 128), aux, dtype=jnp.float32)
        out_ref[...] = jnp.zeros_like(out_ref)

    xb = x_ref[...].astype(jnp.bfloat16)
    h = jax.lax.dot_general(
        xb, w1_ref[0].astype(jnp.bfloat16), (((1,), (0,)), ((), ())),
        preferred_element_type=jnp.float32,
    ) + b1_ref[e, :][None, :]
    h = 0.5 * h * (1.0 + jax.lax.erf(h * 0.7071067811865476))
    y = jax.lax.dot_general(
        h.astype(jnp.bfloat16), w2_ref[0].astype(jnp.bfloat16),
        (((1,), (0,)), ((), ())),
        preferred_element_type=jnp.float32,
    ) + b2_ref[e, :][None, :]
    iota = jax.lax.broadcasted_iota(jnp.int32, (T, E), 1)
    col = jnp.sum(jnp.where(iota == e, gatew_ref[...], 0.0), axis=1)
    out_ref[...] += y * col[:, None]


@jax.jit
def kernel(x, gate_W, gate_b, W1, b1, W2, b2):
    out, aux = pl.pallas_call(
        _moe_kernel,
        grid=(E,),
        in_specs=[
            pl.BlockSpec((T, D), lambda e: (0, 0)),
            pl.BlockSpec((D, E), lambda e: (0, 0)),
            pl.BlockSpec((1, E), lambda e: (0, 0)),
            pl.BlockSpec((1, D, H), lambda e: (e, 0, 0)),
            pl.BlockSpec((E, H), lambda e: (0, 0)),
            pl.BlockSpec((1, H, D), lambda e: (e, 0, 0)),
            pl.BlockSpec((E, D), lambda e: (0, 0)),
        ],
        out_specs=[
            pl.BlockSpec((T, D), lambda e: (0, 0)),
            pl.BlockSpec((8, 128), lambda e: (0, 0)),
        ],
        out_shape=[
            jax.ShapeDtypeStruct((T, D), jnp.float32),
            jax.ShapeDtypeStruct((8, 128), jnp.float32),
        ],
        scratch_shapes=[pltpu.VMEM((T, E), jnp.float32)],
        compiler_params=pltpu.CompilerParams(
            dimension_semantics=("arbitrary",),
        ),
    )(x, gate_W, gate_b.reshape(1, E), W1, b1, W2, b2)
    return out, aux[0, 0]
